# baseline (speedup 1.0000x reference)
import functools

import jax
import jax.numpy as jnp
from jax import lax
from jax.experimental import pallas as pl
from jax.experimental.pallas import tpu as pltpu
from jax.experimental.pallas import tpu_sc as plsc

_NC = 2
_NS = 16
_NW = _NC * _NS


def _sc_gather(n_tot: int, d: int, ch: int):
    n_per_w = n_tot // _NW
    nch = n_per_w // ch
    mesh = plsc.VectorSubcoreMesh(core_axis_name="c", subcore_axis_name="s")

    @functools.partial(
        pl.kernel,
        mesh=mesh,
        compiler_params=pltpu.CompilerParams(use_tc_tiling_on_sc=False),
        out_type=jax.ShapeDtypeStruct((n_tot, d), jnp.float32),
        scratch_types=[
            pltpu.VMEM((n_per_w,), jnp.int32),
            pltpu.VMEM((ch, d), jnp.float32),
            pltpu.SemaphoreType.DMA,
        ],
    )
    def k(idx_hbm, table_hbm, out_hbm, idx_v, rows_v, sem):
        wid = lax.axis_index("s") * _NC + lax.axis_index("c")
        base = wid * n_per_w
        pltpu.sync_copy(idx_hbm.at[pl.ds(base, n_per_w)], idx_v)
        for c in range(nch):
            idx_c = idx_v.at[pl.ds(c * ch, ch)]
            pltpu.async_copy(table_hbm.at[idx_c], rows_v, sem).wait()
            pltpu.sync_copy(rows_v, out_hbm.at[pl.ds(base + c * ch, ch)])

    return k


def _tc_head(n_tot: int, d: int, v: int, r: int):
    nblk = n_tot // r

    def body(tok_ref, pos_ref, w_ref, b_ref, out_ref):
        h = tok_ref[...] + pos_ref[...]
        out_ref[...] = (
            jnp.dot(h, w_ref[...], preferred_element_type=jnp.float32)
            + b_ref[...]
        )

    return pl.pallas_call(
        body,
        grid=(nblk,),
        in_specs=[
            pl.BlockSpec((r, d), lambda i: (i, 0)),
            pl.BlockSpec((r, d), lambda i: (0, 0)),
            pl.BlockSpec((d, v), lambda i: (0, 0)),
            pl.BlockSpec((1, v), lambda i: (0, 0)),
        ],
        out_specs=pl.BlockSpec((r, v), lambda i: (i, 0)),
        out_shape=jax.ShapeDtypeStruct((n_tot, v), jnp.float32),
    )


def kernel(x, tok_table, pos_table, W, b):
    bx, tx = x.shape
    vocab, d = tok_table.shape
    n_tot = bx * tx

    idx = x.reshape(n_tot).astype(jnp.int32)
    tok_emb = _sc_gather(n_tot, d, ch=1600)(idx, tok_table)

    r = 64 * tx
    pos_tiled = jnp.tile(pos_table, (r // tx, 1))
    logits = _tc_head(n_tot, d, vocab, r)(
        tok_emb, pos_tiled, W, b.reshape(1, vocab)
    )
    return logits.reshape(bx, tx, vocab)
